# Initial kernel scaffold; baseline (speedup 1.0000x reference)
import jax
import jax.numpy as jnp
from jax.experimental import pallas as pl


def kernel(node_feats, edge_index, edge_vals, nodes_mask, W0, W1):
    raise NotImplementedError("write your pallas kernel here")


# speedup vs baseline: 2.6429x; 2.6429x over previous
import functools

import jax
import jax.numpy as jnp
from jax import lax
from jax.experimental import pallas as pl
from jax.experimental.pallas import tpu as pltpu
from jax.experimental.pallas import tpu_sc as plsc

N = 10000
E = 320000
D = 128
H = D // 2
L = 16

NC = 2
NS = 16
EPT = E // NS
CHUNK = 80
NCHUNK = EPT // CHUNK
VGRP = CHUNK // L
ROWS_PER_TILE = 624
ROWS_REM = N - NS * ROWS_PER_TILE

_mesh = plsc.VectorSubcoreMesh(core_axis_name="c", subcore_axis_name="s")


@functools.partial(
    pl.kernel,
    out_type=jax.ShapeDtypeStruct((NC, N, H), jnp.float32),
    mesh=_mesh,
    compiler_params=pltpu.CompilerParams(use_tc_tiling_on_sc=False),
    scratch_types=[
        pltpu.VMEM((NCHUNK, CHUNK), jnp.int32),
        pltpu.VMEM((NCHUNK, CHUNK), jnp.int32),
        pltpu.VMEM((EPT,), jnp.float32),
        pltpu.VMEM((CHUNK, H), jnp.float32),
        pltpu.VMEM_SHARED((N, H), jnp.float32),
        pltpu.SemaphoreType.DMA,
    ],
)
def _spmm_sc(hlo_hbm, hhi_hbm, src_hbm, dst_hbm, vals_hbm, zeros_hbm, out_hbm,
             src_v, dst_v, vals_v, rows_v, acc, sem):
    cid = lax.axis_index("c")
    sid = lax.axis_index("s")

    pltpu.sync_copy(src_hbm.at[sid], src_v)
    pltpu.sync_copy(dst_hbm.at[sid], dst_v)
    pltpu.sync_copy(vals_hbm.at[sid], vals_v)

    pltpu.sync_copy(zeros_hbm.at[pl.ds(sid * ROWS_PER_TILE, ROWS_PER_TILE)],
                    acc.at[pl.ds(sid * ROWS_PER_TILE, ROWS_PER_TILE)])

    @pl.when(sid == 0)
    def _():
        pltpu.sync_copy(zeros_hbm.at[pl.ds(NS * ROWS_PER_TILE, ROWS_REM)],
                        acc.at[pl.ds(NS * ROWS_PER_TILE, ROWS_REM)])

    plsc.subcore_barrier()

    def scale_and_scatter(j):
        def group_body(g, _):
            vv16 = vals_v[pl.ds(j * CHUNK + g * L, L)]
            for i in range(L):
                e = g * L + i
                vv = jnp.full((L,), vv16[i], jnp.float32)
                for k in range(H // L):
                    sl = pl.ds(k * L, L)
                    rows_v[e, sl] = rows_v[e, sl] * vv
            return ()

        lax.fori_loop(0, VGRP, group_body, ())
        pltpu.sync_copy(rows_v, acc.at[dst_v.at[j]], add=True)

    def chunk_body_lo(j, _):
        pltpu.async_copy(hlo_hbm.at[src_v.at[j]], rows_v, sem).wait()
        scale_and_scatter(j)
        return ()

    def chunk_body_hi(j, _):
        pltpu.async_copy(hhi_hbm.at[src_v.at[j]], rows_v, sem).wait()
        scale_and_scatter(j)
        return ()

    @pl.when(cid == 0)
    def _():
        lax.fori_loop(0, NCHUNK, chunk_body_lo, ())

    @pl.when(cid == 1)
    def _():
        lax.fori_loop(0, NCHUNK, chunk_body_hi, ())

    plsc.subcore_barrier()
    pltpu.sync_copy(acc.at[pl.ds(sid * ROWS_PER_TILE, ROWS_PER_TILE)],
                    out_hbm.at[cid, pl.ds(sid * ROWS_PER_TILE, ROWS_PER_TILE)])

    @pl.when(sid == 0)
    def _():
        pltpu.sync_copy(acc.at[pl.ds(NS * ROWS_PER_TILE, ROWS_REM)],
                        out_hbm.at[cid, pl.ds(NS * ROWS_PER_TILE, ROWS_REM)])


def _mm_body(x_ref, w_ref, lo_ref, hi_ref):
    h = jnp.dot(x_ref[...], w_ref[...], preferred_element_type=jnp.float32)
    lo_ref[...] = h[:, :H]
    hi_ref[...] = h[:, H:]


def _mm_combine_body(p_ref, w_ref, lo_ref, hi_ref):
    x = jnp.concatenate([jnp.maximum(p_ref[0], 0.0),
                         jnp.maximum(p_ref[1], 0.0)], axis=1)
    h = jnp.dot(x, w_ref[...], preferred_element_type=jnp.float32)
    lo_ref[...] = h[:, :H]
    hi_ref[...] = h[:, H:]


def _combine_body(q_ref, o_ref):
    o_ref[:, :H] = jnp.maximum(q_ref[0], 0.0)
    o_ref[:, H:] = jnp.maximum(q_ref[1], 0.0)


_half_shapes = [jax.ShapeDtypeStruct((N, H), jnp.float32)] * 2
_mm = pl.pallas_call(_mm_body, out_shape=_half_shapes)
_mm_combine = pl.pallas_call(_mm_combine_body, out_shape=_half_shapes)
_combine = pl.pallas_call(
    _combine_body, out_shape=jax.ShapeDtypeStruct((N, D), jnp.float32))


def kernel(node_feats, edge_index, edge_vals, nodes_mask, W0, W1):
    src = edge_index[0].reshape(NS, NCHUNK, CHUNK)
    dst = edge_index[1].reshape(NS, NCHUNK, CHUNK)
    vals = edge_vals.reshape(NS, EPT)
    zeros = jnp.zeros((N, H), jnp.float32)

    h0_lo, h0_hi = _mm(node_feats, W0)
    p = _spmm_sc(h0_lo, h0_hi, src, dst, vals, zeros)
    h1_lo, h1_hi = _mm_combine(p, W1)
    q = _spmm_sc(h1_lo, h1_hi, src, dst, vals, zeros)
    return _combine(q)


# speedup vs baseline: 8.7215x; 3.3000x over previous
import functools

import jax
import jax.numpy as jnp
from jax import lax
from jax.experimental import pallas as pl
from jax.experimental.pallas import tpu as pltpu
from jax.experimental.pallas import tpu_sc as plsc

N = 10000
E = 320000
D = 128
H = D // 2
L = 16

NC = 2
NS = 16
EPT = E // NS
CHUNK = 80
NCHUNK = EPT // CHUNK
VGRP = CHUNK // L
NBUF = 2
ROWS_PER_TILE = 624
ROWS_REM = N - NS * ROWS_PER_TILE

_mesh = plsc.VectorSubcoreMesh(core_axis_name="c", subcore_axis_name="s")


@functools.partial(
    pl.kernel,
    out_type=jax.ShapeDtypeStruct((NC, N, H), jnp.float32),
    mesh=_mesh,
    compiler_params=pltpu.CompilerParams(use_tc_tiling_on_sc=False),
    scratch_types=[
        pltpu.VMEM((NCHUNK, CHUNK), jnp.int32),
        pltpu.VMEM((NCHUNK, CHUNK), jnp.int32),
        pltpu.VMEM((EPT,), jnp.float32),
        [pltpu.VMEM((CHUNK, H), jnp.float32)] * NBUF,
        [pltpu.VMEM((CHUNK, H), jnp.float32)] * NBUF,
        pltpu.VMEM_SHARED((N, H), jnp.float32),
        [pltpu.SemaphoreType.DMA] * NBUF,
        [pltpu.SemaphoreType.DMA] * NBUF,
    ],
)
def _spmm_sc(hlo_hbm, hhi_hbm, src_hbm, dst_hbm, vals_hbm, zeros_hbm, out_hbm,
             src_v, dst_v, vals_v, grow, srow, acc, gsem, ssem):
    cid = lax.axis_index("c")
    sid = lax.axis_index("s")

    pltpu.sync_copy(src_hbm.at[sid], src_v)
    pltpu.sync_copy(dst_hbm.at[sid], dst_v)
    pltpu.sync_copy(vals_hbm.at[sid], vals_v)

    pltpu.sync_copy(zeros_hbm.at[pl.ds(sid * ROWS_PER_TILE, ROWS_PER_TILE)],
                    acc.at[pl.ds(sid * ROWS_PER_TILE, ROWS_PER_TILE)])

    @pl.when(sid == 0)
    def _():
        pltpu.sync_copy(zeros_hbm.at[pl.ds(NS * ROWS_PER_TILE, ROWS_REM)],
                        acc.at[pl.ds(NS * ROWS_PER_TILE, ROWS_REM)])

    plsc.subcore_barrier()

    def run(h_hbm):
        for b in range(NBUF):
            pltpu.async_copy(h_hbm.at[src_v.at[b]], grow[b], gsem[b])

        @pl.loop(0, NCHUNK, step=NBUF)
        def _(j):
            for b in range(NBUF):
                jj = j + b
                pltpu.make_async_copy(h_hbm.at[src_v.at[jj]], grow[b],
                                      gsem[b]).wait()

                @pl.when(jj >= NBUF)
                def _():
                    pltpu.make_async_copy(srow[b], acc.at[dst_v.at[jj]],
                                          ssem[b]).wait()

                for g in range(VGRP):
                    vv16 = vals_v[pl.ds(jj * CHUNK + g * L, L)]
                    for i in range(L):
                        e = g * L + i
                        vv = jnp.full((L,), vv16[i], jnp.float32)
                        for k in range(H // L):
                            sl = pl.ds(k * L, L)
                            srow[b][e, sl] = grow[b][e, sl] * vv

                pltpu.async_copy(srow[b], acc.at[dst_v.at[jj]], ssem[b],
                                 add=True)

                @pl.when(jj + NBUF < NCHUNK)
                def _():
                    pltpu.async_copy(h_hbm.at[src_v.at[jj + NBUF]], grow[b],
                                     gsem[b])

        for b in range(NBUF):
            pltpu.make_async_copy(srow[b], acc.at[dst_v.at[NCHUNK - NBUF + b]],
                                  ssem[b]).wait()

    @pl.when(cid == 0)
    def _():
        run(hlo_hbm)

    @pl.when(cid == 1)
    def _():
        run(hhi_hbm)

    plsc.subcore_barrier()
    pltpu.sync_copy(acc.at[pl.ds(sid * ROWS_PER_TILE, ROWS_PER_TILE)],
                    out_hbm.at[cid, pl.ds(sid * ROWS_PER_TILE, ROWS_PER_TILE)])

    @pl.when(sid == 0)
    def _():
        pltpu.sync_copy(acc.at[pl.ds(NS * ROWS_PER_TILE, ROWS_REM)],
                        out_hbm.at[cid, pl.ds(NS * ROWS_PER_TILE, ROWS_REM)])


def _mm_body(x_ref, w_ref, lo_ref, hi_ref):
    h = jnp.dot(x_ref[...], w_ref[...], preferred_element_type=jnp.float32)
    lo_ref[...] = h[:, :H]
    hi_ref[...] = h[:, H:]


def _mm_combine_body(p_ref, w_ref, lo_ref, hi_ref):
    x = jnp.concatenate([jnp.maximum(p_ref[0], 0.0),
                         jnp.maximum(p_ref[1], 0.0)], axis=1)
    h = jnp.dot(x, w_ref[...], preferred_element_type=jnp.float32)
    lo_ref[...] = h[:, :H]
    hi_ref[...] = h[:, H:]


def _combine_body(q_ref, o_ref):
    o_ref[:, :H] = jnp.maximum(q_ref[0], 0.0)
    o_ref[:, H:] = jnp.maximum(q_ref[1], 0.0)


_half_shapes = [jax.ShapeDtypeStruct((N, H), jnp.float32)] * 2
_mm = pl.pallas_call(_mm_body, out_shape=_half_shapes)
_mm_combine = pl.pallas_call(_mm_combine_body, out_shape=_half_shapes)
_combine = pl.pallas_call(
    _combine_body, out_shape=jax.ShapeDtypeStruct((N, D), jnp.float32))


def kernel(node_feats, edge_index, edge_vals, nodes_mask, W0, W1):
    src = edge_index[0].reshape(NS, NCHUNK, CHUNK)
    dst = edge_index[1].reshape(NS, NCHUNK, CHUNK)
    vals = edge_vals.reshape(NS, EPT)
    zeros = jnp.zeros((N, H), jnp.float32)

    h0_lo, h0_hi = _mm(node_feats, W0)
    p = _spmm_sc(h0_lo, h0_hi, src, dst, vals, zeros)
    h1_lo, h1_hi = _mm_combine(p, W1)
    q = _spmm_sc(h1_lo, h1_hi, src, dst, vals, zeros)
    return _combine(q)


# speedup vs baseline: 8.7283x; 1.0008x over previous
import functools

import jax
import jax.numpy as jnp
from jax import lax
from jax.experimental import pallas as pl
from jax.experimental.pallas import tpu as pltpu
from jax.experimental.pallas import tpu_sc as plsc

N = 10000
E = 320000
D = 128
H = D // 2
L = 16

NC = 2
NS = 16
EPT = E // NS
CHUNK = 80
NCHUNK = EPT // CHUNK
VGRP = CHUNK // L
NBUF = 2
ROWS_PER_TILE = 624
ROWS_REM = N - NS * ROWS_PER_TILE

_mesh = plsc.VectorSubcoreMesh(core_axis_name="c", subcore_axis_name="s")

_SPLAT_DNUMS = lax.GatherDimensionNumbers(
    offset_dims=(), collapsed_slice_dims=(0,), start_index_map=(0,))


def _splat(v, lane):
    idx = jnp.full((L, 1), lane, jnp.int32)
    return lax.gather(v, idx, _SPLAT_DNUMS, (1,),
                      mode=lax.GatherScatterMode.PROMISE_IN_BOUNDS)


@functools.partial(
    pl.kernel,
    out_type=jax.ShapeDtypeStruct((NC, N, H), jnp.float32),
    mesh=_mesh,
    compiler_params=pltpu.CompilerParams(use_tc_tiling_on_sc=False),
    scratch_types=[
        pltpu.VMEM((NCHUNK, CHUNK), jnp.int32),
        pltpu.VMEM((NCHUNK, CHUNK), jnp.int32),
        pltpu.VMEM((EPT,), jnp.float32),
        [pltpu.VMEM((CHUNK, H), jnp.float32)] * NBUF,
        [pltpu.VMEM((CHUNK, H), jnp.float32)] * NBUF,
        pltpu.VMEM_SHARED((N, H), jnp.float32),
        [pltpu.SemaphoreType.DMA] * NBUF,
        [pltpu.SemaphoreType.DMA] * NBUF,
    ],
)
def _spmm_sc(hlo_hbm, hhi_hbm, src_hbm, dst_hbm, vals_hbm, zeros_hbm, out_hbm,
             src_v, dst_v, vals_v, grow, srow, acc, gsem, ssem):
    cid = lax.axis_index("c")
    sid = lax.axis_index("s")

    pltpu.sync_copy(src_hbm.at[sid], src_v)
    pltpu.sync_copy(dst_hbm.at[sid], dst_v)
    pltpu.sync_copy(vals_hbm.at[sid], vals_v)

    pltpu.sync_copy(zeros_hbm.at[pl.ds(sid * ROWS_PER_TILE, ROWS_PER_TILE)],
                    acc.at[pl.ds(sid * ROWS_PER_TILE, ROWS_PER_TILE)])

    @pl.when(sid == 0)
    def _():
        pltpu.sync_copy(zeros_hbm.at[pl.ds(NS * ROWS_PER_TILE, ROWS_REM)],
                        acc.at[pl.ds(NS * ROWS_PER_TILE, ROWS_REM)])

    plsc.subcore_barrier()

    def run(h_hbm):
        for b in range(NBUF):
            pltpu.async_copy(h_hbm.at[src_v.at[b]], grow[b], gsem[b])

        @pl.loop(0, NCHUNK, step=NBUF)
        def _(j):
            for b in range(NBUF):
                jj = j + b
                pltpu.make_async_copy(h_hbm.at[src_v.at[jj]], grow[b],
                                      gsem[b]).wait()

                @pl.when(jj >= NBUF)
                def _():
                    pltpu.make_async_copy(srow[b], acc.at[dst_v.at[jj]],
                                          ssem[b]).wait()

                base = jj * CHUNK
                for g in range(VGRP):
                    v = vals_v[pl.ds(base + g * L, L)]
                    for r in range(L):
                        row = g * L + r
                        s = _splat(v, r)
                        for q in range(H // L):
                            sl = pl.ds(q * L, L)
                            srow[b][row, sl] = grow[b][row, sl] * s

                pltpu.async_copy(srow[b], acc.at[dst_v.at[jj]], ssem[b],
                                 add=True)

                @pl.when(jj + NBUF < NCHUNK)
                def _():
                    pltpu.async_copy(h_hbm.at[src_v.at[jj + NBUF]], grow[b],
                                     gsem[b])

        for b in range(NBUF):
            pltpu.make_async_copy(srow[b], acc.at[dst_v.at[NCHUNK - NBUF + b]],
                                  ssem[b]).wait()

    @pl.when(cid == 0)
    def _():
        run(hlo_hbm)

    @pl.when(cid == 1)
    def _():
        run(hhi_hbm)

    plsc.subcore_barrier()
    pltpu.sync_copy(acc.at[pl.ds(sid * ROWS_PER_TILE, ROWS_PER_TILE)],
                    out_hbm.at[cid, pl.ds(sid * ROWS_PER_TILE, ROWS_PER_TILE)])

    @pl.when(sid == 0)
    def _():
        pltpu.sync_copy(acc.at[pl.ds(NS * ROWS_PER_TILE, ROWS_REM)],
                        out_hbm.at[cid, pl.ds(NS * ROWS_PER_TILE, ROWS_REM)])


def _mm_body(x_ref, w_ref, lo_ref, hi_ref):
    h = jnp.dot(x_ref[...], w_ref[...], preferred_element_type=jnp.float32)
    lo_ref[...] = h[:, :H]
    hi_ref[...] = h[:, H:]


def _mm_combine_body(p_ref, w_ref, lo_ref, hi_ref):
    x = jnp.concatenate([jnp.maximum(p_ref[0], 0.0),
                         jnp.maximum(p_ref[1], 0.0)], axis=1)
    h = jnp.dot(x, w_ref[...], preferred_element_type=jnp.float32)
    lo_ref[...] = h[:, :H]
    hi_ref[...] = h[:, H:]


def _combine_body(q_ref, o_ref):
    o_ref[:, :H] = jnp.maximum(q_ref[0], 0.0)
    o_ref[:, H:] = jnp.maximum(q_ref[1], 0.0)


_half_shapes = [jax.ShapeDtypeStruct((N, H), jnp.float32)] * 2
_mm = pl.pallas_call(_mm_body, out_shape=_half_shapes)
_mm_combine = pl.pallas_call(_mm_combine_body, out_shape=_half_shapes)
_combine = pl.pallas_call(
    _combine_body, out_shape=jax.ShapeDtypeStruct((N, D), jnp.float32))


def kernel(node_feats, edge_index, edge_vals, nodes_mask, W0, W1):
    src = edge_index[0].reshape(NS, NCHUNK, CHUNK)
    dst = edge_index[1].reshape(NS, NCHUNK, CHUNK)
    vals = edge_vals.reshape(NS, EPT)
    zeros = jnp.zeros((N, H), jnp.float32)

    h0_lo, h0_hi = _mm(node_feats, W0)
    p = _spmm_sc(h0_lo, h0_hi, src, dst, vals, zeros)
    h1_lo, h1_hi = _mm_combine(p, W1)
    q = _spmm_sc(h1_lo, h1_hi, src, dst, vals, zeros)
    return _combine(q)
